# baseline (speedup 1.0000x reference)
import functools

import jax
import jax.numpy as jnp
from jax import lax
from jax.experimental import pallas as pl
from jax.experimental.pallas import tpu as pltpu
from jax.experimental.pallas import tpu_sc as plsc

_B = 16384
_D = 512
_NW = 32
_ROWS = 4 * _B
_BPW = _ROWS // _NW
_CHUNK = 64
_NCHUNK = _BPW // _CHUNK

_mesh = plsc.VectorSubcoreMesh(core_axis_name="c", subcore_axis_name="s")


@functools.partial(
    pl.kernel,
    mesh=_mesh,
    out_type=jax.ShapeDtypeStruct((_ROWS, _D), jnp.float32),
    scratch_types=[
        pltpu.VMEM((_BPW,), jnp.int32),
        pltpu.VMEM((_CHUNK, _D), jnp.float32),
        pltpu.VMEM((_CHUNK, _D), jnp.float32),
        pltpu.SemaphoreType.DMA,
        pltpu.SemaphoreType.DMA,
    ],
)
def _gather_all(table_hbm, idx_hbm, out_hbm, idx_v, buf0, buf1, gsem, ssem):
    bufs = (buf0, buf1)
    wid = lax.axis_index("s") * 2 + lax.axis_index("c")
    base = wid * _BPW
    pltpu.sync_copy(idx_hbm.at[pl.ds(base, _BPW)], idx_v)

    def fire_gather(g, b):
        pltpu.async_copy(
            table_hbm.at[idx_v.at[pl.ds(g * _CHUNK, _CHUNK)]], bufs[b], gsem
        )

    def wait_gather(b):
        pltpu.make_async_copy(table_hbm.at[pl.ds(0, _CHUNK)], bufs[b], gsem).wait()

    def fire_scatter(g, b):
        pltpu.async_copy(bufs[b], out_hbm.at[pl.ds(base + g * _CHUNK, _CHUNK)], ssem)

    def wait_scatter(b):
        pltpu.make_async_copy(bufs[b], out_hbm.at[pl.ds(base, _CHUNK)], ssem).wait()

    fire_gather(0, 0)
    fire_gather(1, 1)

    def body(i, carry):
        g = i * 2
        for b in range(2):
            wait_gather(b)
            fire_scatter(g + b, b)
            wait_scatter(b)
            fire_gather(g + b + 2, b)
        return carry

    lax.fori_loop(0, _NCHUNK // 2 - 1, body, 0)

    g_last = _NCHUNK - 2
    for b in range(2):
        wait_gather(b)
        fire_scatter(g_last + b, b)
    for b in range(2):
        wait_scatter(b)


def kernel(hour, day, weekday, month, W_hour, W_day, W_weekday, W_month):
    table = jnp.concatenate([W_hour, W_day, W_weekday, W_month], axis=0)
    idx = jnp.stack(
        [
            hour.astype(jnp.int32),
            day.astype(jnp.int32) + 24,
            weekday.astype(jnp.int32) + 55,
            month.astype(jnp.int32) + 62,
        ],
        axis=1,
    ).reshape(_ROWS)
    out = _gather_all(table, idx)
    return out.reshape(_B, 4 * _D)


# speedup vs baseline: 1.6545x; 1.6545x over previous
import functools

import jax
import jax.numpy as jnp
from jax import lax
from jax.experimental import pallas as pl
from jax.experimental.pallas import tpu as pltpu
from jax.experimental.pallas import tpu_sc as plsc

_B = 16384
_D = 512
_NW = 32
_ROWS = 4 * _B
_BPW = _ROWS // _NW
_CHUNK = 64
_NCHUNK = _BPW // _CHUNK

_mesh = plsc.VectorSubcoreMesh(core_axis_name="c", subcore_axis_name="s")


_VROWS = 74


@functools.partial(
    pl.kernel,
    mesh=_mesh,
    out_type=jax.ShapeDtypeStruct((_ROWS, _D), jnp.float32),
    scratch_types=[
        pltpu.VMEM((_BPW,), jnp.int32),
        pltpu.VMEM((_CHUNK, _D), jnp.float32),
        pltpu.VMEM((_CHUNK, _D), jnp.float32),
        pltpu.SemaphoreType.DMA,
        pltpu.SemaphoreType.DMA,
    ],
)
def _gather_all(table_hbm, idx_hbm, out_hbm, idx_v, buf0, buf1, gsem, ssem):
    bufs = (buf0, buf1)
    sid = lax.axis_index("s")
    wid = sid * 2 + lax.axis_index("c")
    base = wid * _BPW

    pltpu.sync_copy(idx_hbm.at[pl.ds(base, _BPW)], idx_v)

    def fire_gather(g, b):
        pltpu.async_copy(
            table_hbm.at[idx_v.at[pl.ds(g * _CHUNK, _CHUNK)]], bufs[b], gsem
        )

    def wait_gather(b):
        pltpu.make_async_copy(table_hbm.at[pl.ds(0, _CHUNK)], bufs[b], gsem).wait()

    def fire_scatter(g, b):
        pltpu.async_copy(bufs[b], out_hbm.at[pl.ds(base + g * _CHUNK, _CHUNK)], ssem)

    def wait_scatter(b):
        pltpu.make_async_copy(bufs[b], out_hbm.at[pl.ds(base, _CHUNK)], ssem).wait()

    fire_gather(0, 0)
    fire_gather(1, 1)

    def body(i, carry):
        g = i * 2
        for b in range(2):
            wait_gather(b)
            fire_scatter(g + b, b)
            wait_scatter(b)
            fire_gather(g + b + 2, b)
        return carry

    lax.fori_loop(0, _NCHUNK // 2 - 1, body, 0)

    g_last = _NCHUNK - 2
    for b in range(2):
        wait_gather(b)
        fire_scatter(g_last + b, b)
    for b in range(2):
        wait_scatter(b)


def kernel(hour, day, weekday, month, W_hour, W_day, W_weekday, W_month):
    table = jnp.concatenate([W_hour, W_day, W_weekday, W_month], axis=0)
    table_rep = jnp.tile(table, (_NW, 1))
    idx = jnp.stack(
        [
            hour.astype(jnp.int32),
            day.astype(jnp.int32) + 24,
            weekday.astype(jnp.int32) + 55,
            month.astype(jnp.int32) + 62,
        ],
        axis=1,
    ).reshape(_ROWS)
    idx = idx + (jnp.arange(_ROWS, dtype=jnp.int32) // _BPW) * _VROWS
    out = _gather_all(table_rep, idx)
    return out.reshape(_B, 4 * _D)
